# Initial kernel scaffold; baseline (speedup 1.0000x reference)
import functools

import jax
import jax.numpy as jnp
from jax import lax
from jax.experimental import pallas as pl
from jax.experimental.pallas import tpu as pltpu
from jax.experimental.pallas import tpu_sc as plsc

_N = 10000
_E = 320000
_NC = 2
_NS = 16
_NW = _NC * _NS


def _make_spmm(F, K=80):
    T = _E // _NW
    n_chunks = T // K
    rows_per_tile = _N // _NS
    ZR = 125
    n_zero = rows_per_tile // ZR
    FL = F // 16

    mesh = plsc.VectorSubcoreMesh(core_axis_name="c", subcore_axis_name="s")

    @functools.partial(
        pl.kernel,
        out_type=jax.ShapeDtypeStruct((_NC, _N, F), jnp.float32),
        mesh=mesh,
        scratch_types=[
            pltpu.VMEM((K,), jnp.int32),
            pltpu.VMEM((K,), jnp.int32),
            pltpu.VMEM((K,), jnp.float32),
            pltpu.VMEM((K, F), jnp.float32),
            pltpu.VMEM((ZR, F), jnp.float32),
            pltpu.VMEM_SHARED((_N, F), jnp.float32),
            pltpu.SemaphoreType.DMA,
        ],
    )
    def spmm(src_hbm, dst_hbm, val_hbm, h_hbm, out_hbm,
             src_v, dst_v, val_v, rows_v, zbuf, acc, sem):
        c = lax.axis_index("c")
        s = lax.axis_index("s")
        wid = s * _NC + c

        def zrow(r, _):
            for f in range(FL):
                zbuf[r, pl.ds(16 * f, 16)] = jnp.zeros((16,), jnp.float32)
            return 0
        lax.fori_loop(0, ZR, zrow, 0)
        for j in range(n_zero):
            pltpu.sync_copy(zbuf, acc.at[pl.ds(s * rows_per_tile + j * ZR, ZR)])
        plsc.subcore_barrier()

        def chunk(nc_, _):
            base = wid * T + nc_ * K
            pltpu.sync_copy(src_hbm.at[pl.ds(base, K)], src_v)
            pltpu.sync_copy(dst_hbm.at[pl.ds(base, K)], dst_v)
            pltpu.sync_copy(val_hbm.at[pl.ds(base, K)], val_v)
            pltpu.async_copy(h_hbm.at[src_v], rows_v, sem).wait()

            def scale(e, _):
                vb = plsc.load_gather(val_v, [jnp.full((16,), e, jnp.int32)])
                for f in range(FL):
                    sl = pl.ds(16 * f, 16)
                    rows_v[e, sl] = rows_v[e, sl] * vb
                return 0
            lax.fori_loop(0, K, scale, 0)

            pltpu.sync_copy(rows_v, acc.at[dst_v], add=True)
            return 0
        lax.fori_loop(0, n_chunks, chunk, 0)

        plsc.subcore_barrier()
        pltpu.sync_copy(
            acc.at[pl.ds(s * rows_per_tile, rows_per_tile)],
            out_hbm.at[c, pl.ds(s * rows_per_tile, rows_per_tile)],
        )

    return spmm


_spmm128 = _make_spmm(128)
_spmm16 = _make_spmm(16)


_BM = 1250


def _mm1_body(x_ref, w_ref, o_ref):
    o_ref[...] = jnp.dot(x_ref[...], w_ref[...],
                         preferred_element_type=jnp.float32)


def _mm1(x, W1):
    M, Kd = x.shape
    Nd = W1.shape[1]
    return pl.pallas_call(
        _mm1_body,
        grid=(M // _BM,),
        in_specs=[
            pl.BlockSpec((_BM, Kd), lambda i: (i, 0)),
            pl.BlockSpec((Kd, Nd), lambda i: (0, 0)),
        ],
        out_specs=pl.BlockSpec((_BM, Nd), lambda i: (i, 0)),
        out_shape=jax.ShapeDtypeStruct((M, Nd), jnp.float32),
    )(x, W1)


def _mm2_body(p_ref, b_ref, w_ref, o_ref):
    h = p_ref[0] + p_ref[1] + b_ref[...]
    h = jnp.maximum(h, 0.0)
    o_ref[...] = jnp.dot(h, w_ref[...], preferred_element_type=jnp.float32)


def _mm2(p, b1, W2):
    M = p.shape[1]
    Kd = p.shape[2]
    Nd = W2.shape[1]
    return pl.pallas_call(
        _mm2_body,
        grid=(M // _BM,),
        in_specs=[
            pl.BlockSpec((2, _BM, Kd), lambda i: (0, i, 0)),
            pl.BlockSpec((1, Kd), lambda i: (0, 0)),
            pl.BlockSpec((Kd, Nd), lambda i: (0, 0)),
        ],
        out_specs=pl.BlockSpec((_BM, Nd), lambda i: (i, 0)),
        out_shape=jax.ShapeDtypeStruct((M, Nd), jnp.float32),
    )(p, b1.reshape(1, Kd), W2)


def _merge_body(q_ref, b_ref, o_ref):
    o_ref[...] = q_ref[0] + q_ref[1] + b_ref[...]


def _merge(q, b2):
    M = q.shape[1]
    Nd = q.shape[2]
    return pl.pallas_call(
        _merge_body,
        grid=(M // _BM,),
        in_specs=[
            pl.BlockSpec((2, _BM, Nd), lambda i: (0, i, 0)),
            pl.BlockSpec((1, Nd), lambda i: (0, 0)),
        ],
        out_specs=pl.BlockSpec((_BM, Nd), lambda i: (i, 0)),
        out_shape=jax.ShapeDtypeStruct((M, Nd), jnp.float32),
    )(q, b2.reshape(1, Nd))


def kernel(x, edge_index, adj_values, W1, b1, W2, b2):
    src = edge_index[0]
    dst = edge_index[1]
    h = _mm1(x, W1)
    p = _spmm128(src, dst, adj_values, h)
    z = _mm2(p, b1, W2)
    q = _spmm16(src, dst, adj_values, z)
    return _merge(q, b2)


# speedup vs baseline: 4.1142x; 4.1142x over previous
import functools

import jax
import jax.numpy as jnp
from jax import lax
from jax.experimental import pallas as pl
from jax.experimental.pallas import tpu as pltpu
from jax.experimental.pallas import tpu_sc as plsc

def _lane_splat(vec, lane):
    idx = jnp.full((16, 1), lane, jnp.int32)
    dn = lax.GatherDimensionNumbers(
        offset_dims=(), collapsed_slice_dims=(0,), start_index_map=(0,))
    return lax.gather(vec, idx, dn, (1,),
                      mode=lax.GatherScatterMode.PROMISE_IN_BOUNDS)


_N = 10000
_E = 320000
_NC = 2
_NS = 16
_NW = _NC * _NS


def _make_spmm(F, K=80):
    T = _E // _NW
    n_chunks = T // K
    WB_TILES = 10
    rows_per_tile = _N // WB_TILES
    ZR = 200
    n_zero = rows_per_tile // ZR
    FL = F // 16

    mesh = plsc.VectorSubcoreMesh(core_axis_name="c", subcore_axis_name="s",
                                  num_cores=_NC, num_subcores=_NS)

    @functools.partial(
        pl.kernel,
        out_type=jax.ShapeDtypeStruct((_NC, _N, F), jnp.float32),
        mesh=mesh,
        scratch_types=[
            pltpu.VMEM((K,), jnp.int32),
            pltpu.VMEM((K,), jnp.int32),
            pltpu.VMEM((K,), jnp.float32),
            pltpu.VMEM((K, F), jnp.float32),
            pltpu.VMEM((ZR, F), jnp.float32),
            pltpu.VMEM_SHARED((_N, F), jnp.float32),
            pltpu.SemaphoreType.DMA,
        ],
    )
    def spmm(src_hbm, dst_hbm, val_hbm, h_hbm, out_hbm,
             src_v, dst_v, val_v, rows_v, zbuf, acc, sem):
        c = lax.axis_index("c")
        s = lax.axis_index("s")
        wid = s * _NC + c

        @pl.when(s < WB_TILES)
        def _():
            def zrow(r, _):
                for f in range(FL):
                    zbuf[r, pl.ds(16 * f, 16)] = jnp.zeros((16,), jnp.float32)
                return 0
            lax.fori_loop(0, ZR, zrow, 0)
            row0 = pl.multiple_of(s * rows_per_tile, 8)
            for j in range(n_zero):
                pltpu.sync_copy(zbuf, acc.at[pl.ds(row0 + j * ZR, ZR)])
        plsc.subcore_barrier()

        def chunk(nc_, _):
            base = pl.multiple_of(wid * T + nc_ * K, 8)
            pltpu.sync_copy(src_hbm.at[pl.ds(base, K)], src_v)
            pltpu.sync_copy(dst_hbm.at[pl.ds(base, K)], dst_v)
            pltpu.sync_copy(val_hbm.at[pl.ds(base, K)], val_v)
            pltpu.async_copy(h_hbm.at[src_v], rows_v, sem).wait()

            def scale(g, _):
                valg = val_v[pl.ds(16 * g, 16)]
                for l in range(16):
                    vb = _lane_splat(valg, l)
                    e = 16 * g + l
                    for f in range(FL):
                        sl = pl.ds(16 * f, 16)
                        rows_v[e, sl] = rows_v[e, sl] * vb
                return 0
            lax.fori_loop(0, K // 16, scale, 0)

            pltpu.sync_copy(rows_v, acc.at[dst_v], add=True)
            return 0
        lax.fori_loop(0, n_chunks, chunk, 0)

        plsc.subcore_barrier()

        @pl.when(s < WB_TILES)
        def _():
            row0 = pl.multiple_of(s * rows_per_tile, 8)
            pltpu.sync_copy(
                acc.at[pl.ds(row0, rows_per_tile)],
                out_hbm.at[c, pl.ds(row0, rows_per_tile)],
            )

    return spmm


_spmm128 = _make_spmm(128)


_BM = 1000


def _mm1_body(x_ref, w_ref, o_ref):
    o_ref[...] = jnp.dot(x_ref[...], w_ref[...],
                         preferred_element_type=jnp.float32)


def _mm1(x, W1):
    M, Kd = x.shape
    Nd = W1.shape[1]
    return pl.pallas_call(
        _mm1_body,
        grid=(M // _BM,),
        in_specs=[
            pl.BlockSpec((_BM, Kd), lambda i: (i, 0)),
            pl.BlockSpec((Kd, Nd), lambda i: (0, 0)),
        ],
        out_specs=pl.BlockSpec((_BM, Nd), lambda i: (i, 0)),
        out_shape=jax.ShapeDtypeStruct((M, Nd), jnp.float32),
    )(x, W1)


def _relu_merge_body(p_ref, b_ref, o_ref):
    o_ref[...] = jnp.maximum(p_ref[0] + p_ref[1] + b_ref[...], 0.0)


def _relu_merge(p, b1):
    M = p.shape[1]
    Kd = p.shape[2]
    return pl.pallas_call(
        _relu_merge_body,
        grid=(M // _BM,),
        in_specs=[
            pl.BlockSpec((2, _BM, Kd), lambda i: (0, i, 0)),
            pl.BlockSpec((1, Kd), lambda i: (0, 0)),
        ],
        out_specs=pl.BlockSpec((_BM, Kd), lambda i: (i, 0)),
        out_shape=jax.ShapeDtypeStruct((M, Kd), jnp.float32),
    )(p, b1.reshape(1, Kd))


def _mm2_body(g_ref, w_ref, b_ref, o_ref):
    t = g_ref[0] + g_ref[1]
    o_ref[...] = jnp.dot(t, w_ref[...],
                         preferred_element_type=jnp.float32) + b_ref[...]


def _mm2(g, W2, b2):
    M = g.shape[1]
    Kd = g.shape[2]
    Nd = W2.shape[1]
    return pl.pallas_call(
        _mm2_body,
        grid=(M // _BM,),
        in_specs=[
            pl.BlockSpec((2, _BM, Kd), lambda i: (0, i, 0)),
            pl.BlockSpec((Kd, Nd), lambda i: (0, 0)),
            pl.BlockSpec((1, Nd), lambda i: (0, 0)),
        ],
        out_specs=pl.BlockSpec((_BM, Nd), lambda i: (i, 0)),
        out_shape=jax.ShapeDtypeStruct((M, Nd), jnp.float32),
    )(g, W2, b2.reshape(1, Nd))


def kernel(x, edge_index, adj_values, W1, b1, W2, b2):
    src = edge_index[0]
    dst = edge_index[1]
    h = _mm1(x, W1)
    p = _spmm128(src, dst, adj_values, h)
    z = _relu_merge(p, b1)
    g = _spmm128(src, dst, adj_values, z)
    return _mm2(g, W2, b2)


# speedup vs baseline: 10.2489x; 2.4911x over previous
import functools

import jax
import jax.numpy as jnp
from jax import lax
from jax.experimental import pallas as pl
from jax.experimental.pallas import tpu as pltpu
from jax.experimental.pallas import tpu_sc as plsc

def _lane_splat(vec, lane):
    idx = jnp.full((16, 1), lane, jnp.int32)
    dn = lax.GatherDimensionNumbers(
        offset_dims=(), collapsed_slice_dims=(0,), start_index_map=(0,))
    return lax.gather(vec, idx, dn, (1,),
                      mode=lax.GatherScatterMode.PROMISE_IN_BOUNDS)


_N = 10000
_E = 320000
_NC = 2
_NS = 16
_NW = _NC * _NS


_K = 80
_NCH = 125


def _make_spmm(F):
    K = _K
    n_chunks = _NCH
    WB_TILES = 10
    rows_per_tile = _N // WB_TILES
    FL = F // 16

    mesh = plsc.VectorSubcoreMesh(core_axis_name="c", subcore_axis_name="s",
                                  num_cores=_NC, num_subcores=_NS)

    @functools.partial(
        pl.kernel,
        out_type=jax.ShapeDtypeStruct((_NC, _N, F), jnp.float32),
        mesh=mesh,
        scratch_types=[
            pltpu.VMEM((n_chunks * K,), jnp.int32),
            pltpu.VMEM((n_chunks * K,), jnp.int32),
            pltpu.VMEM((n_chunks * K,), jnp.float32),
            pltpu.VMEM((K, F), jnp.float32),
            pltpu.VMEM((K, F), jnp.float32),
            pltpu.VMEM_SHARED((_N, F), jnp.float32),
            pltpu.SemaphoreType.DMA,
            pltpu.SemaphoreType.DMA,
        ],
    )
    def spmm(src_hbm, dst_hbm, val_hbm, h_hbm, zero_hbm, out_hbm,
             src_a, dst_a, val_a, rows0, rows1, acc, sem0, sem1):
        c = lax.axis_index("c")
        s = lax.axis_index("s")
        wid = s * _NC + c

        pltpu.sync_copy(src_hbm.at[wid], src_a)
        pltpu.sync_copy(dst_hbm.at[wid], dst_a)
        pltpu.sync_copy(val_hbm.at[wid], val_a)

        @pl.when(s < WB_TILES)
        def _():
            row0_ = pl.multiple_of(s * rows_per_tile, 8)
            pltpu.sync_copy(zero_hbm.at[pl.ds(row0_, rows_per_tile)],
                            acc.at[pl.ds(row0_, rows_per_tile)])
        plsc.subcore_barrier()

        def gather_start(j, buf, sem):
            pltpu.async_copy(h_hbm.at[src_a.at[pl.ds(j * K, K)]], buf, sem)

        def gather_wait(j, buf, sem):
            pltpu.make_async_copy(
                h_hbm.at[src_a.at[pl.ds(j * K, K)]], buf, sem).wait()

        def scale_scatter(j, buf):
            def scale(g, _):
                valg = val_a[pl.ds(j * K + 16 * g, 16)]
                for l in range(16):
                    vb = _lane_splat(valg, l)
                    e = 16 * g + l
                    for f in range(FL):
                        sl = pl.ds(16 * f, 16)
                        buf[e, sl] = buf[e, sl] * vb
                return 0
            lax.fori_loop(0, K // 16, scale, 0)
            pltpu.sync_copy(buf, acc.at[dst_a.at[pl.ds(j * K, K)]], add=True)

        gather_start(0, rows0, sem0)

        def body(jj, _):
            j0 = 2 * jj
            j1 = j0 + 1
            gather_start(j1, rows1, sem1)
            gather_wait(j0, rows0, sem0)
            scale_scatter(j0, rows0)
            gather_start(j0 + 2, rows0, sem0)
            gather_wait(j1, rows1, sem1)
            scale_scatter(j1, rows1)
            return 0
        lax.fori_loop(0, n_chunks // 2, body, 0)
        gather_wait(n_chunks - 1, rows0, sem0)
        scale_scatter(n_chunks - 1, rows0)

        plsc.subcore_barrier()

        @pl.when(s < WB_TILES)
        def _():
            row0_ = pl.multiple_of(s * rows_per_tile, 8)
            pltpu.sync_copy(
                acc.at[pl.ds(row0_, rows_per_tile)],
                out_hbm.at[c, pl.ds(row0_, rows_per_tile)],
            )

    return spmm


_spmm128 = _make_spmm(128)


def _prep_edges(a):
    return a.reshape(_NW, _E // _NW)


_BM = 1000


def _mm1_body(x_ref, w_ref, o_ref):
    o_ref[...] = jnp.dot(x_ref[...], w_ref[...],
                         preferred_element_type=jnp.float32)


def _mm1(x, W1):
    M, Kd = x.shape
    Nd = W1.shape[1]
    return pl.pallas_call(
        _mm1_body,
        grid=(M // _BM,),
        in_specs=[
            pl.BlockSpec((_BM, Kd), lambda i: (i, 0)),
            pl.BlockSpec((Kd, Nd), lambda i: (0, 0)),
        ],
        out_specs=pl.BlockSpec((_BM, Nd), lambda i: (i, 0)),
        out_shape=jax.ShapeDtypeStruct((M, Nd), jnp.float32),
    )(x, W1)


def _relu_merge_body(p_ref, b_ref, o_ref):
    o_ref[...] = jnp.maximum(p_ref[0] + p_ref[1] + b_ref[...], 0.0)


def _relu_merge(p, b1):
    M = p.shape[1]
    Kd = p.shape[2]
    return pl.pallas_call(
        _relu_merge_body,
        grid=(M // _BM,),
        in_specs=[
            pl.BlockSpec((2, _BM, Kd), lambda i: (0, i, 0)),
            pl.BlockSpec((1, Kd), lambda i: (0, 0)),
        ],
        out_specs=pl.BlockSpec((_BM, Kd), lambda i: (i, 0)),
        out_shape=jax.ShapeDtypeStruct((M, Kd), jnp.float32),
    )(p, b1.reshape(1, Kd))


def _mm2_body(g_ref, w_ref, b_ref, o_ref):
    t = g_ref[0] + g_ref[1]
    o_ref[...] = jnp.dot(t, w_ref[...],
                         preferred_element_type=jnp.float32) + b_ref[...]


def _mm2(g, W2, b2):
    M = g.shape[1]
    Kd = g.shape[2]
    Nd = W2.shape[1]
    return pl.pallas_call(
        _mm2_body,
        grid=(M // _BM,),
        in_specs=[
            pl.BlockSpec((2, _BM, Kd), lambda i: (0, i, 0)),
            pl.BlockSpec((Kd, Nd), lambda i: (0, 0)),
            pl.BlockSpec((1, Nd), lambda i: (0, 0)),
        ],
        out_specs=pl.BlockSpec((_BM, Nd), lambda i: (i, 0)),
        out_shape=jax.ShapeDtypeStruct((M, Nd), jnp.float32),
    )(g, W2, b2.reshape(1, Nd))


def kernel(x, edge_index, adj_values, W1, b1, W2, b2):
    src = _prep_edges(edge_index[0])
    dst = _prep_edges(edge_index[1])
    adj_values = _prep_edges(adj_values)
    zero = jnp.zeros((_N, 128), jnp.float32)
    h = _mm1(x, W1)
    p = _spmm128(src, dst, adj_values, h, zero)
    z = _relu_merge(p, b1)
    g = _spmm128(src, dst, adj_values, z, zero)
    return _mm2(g, W2, b2)
